# Initial kernel scaffold; baseline (speedup 1.0000x reference)
import functools

import jax
import jax.numpy as jnp
from jax import lax
from jax.experimental import pallas as pl
from jax.experimental.pallas import tpu as pltpu
from jax.experimental.pallas import tpu_sc as plsc

N_NODES = 10000
N_EDGES = 320000
NODE_DIM = 128
Z_ROWS = 256

NC = 2
NS = 16
L = 16

NODES_PAD = 10240
NODES_PER_TILE = NODES_PAD // (NC * NS)
SUB = 80
N_SUB = NODES_PER_TILE // SUB
EDGES_PER_TILE = N_EDGES // NS


def _body(x_hbm, edges_hbm, zin_hbm, zout_hbm, out_hbm,
          ebuf0, ebuf1, hist_in, hist_out, shared, cbuf,
          idx_in, idx_out, xbuf, zin_buf, zout_buf,
          sem_x, sem_zi, sem_zo):
    c = lax.axis_index("c")
    s = lax.axis_index("s")

    zeros = jnp.zeros((L,), jnp.int32)
    ones = jnp.ones((L,), jnp.int32)

    def zero_body(i, _):
        hist_in[pl.ds(i * L, L)] = zeros
        hist_out[pl.ds(i * L, L)] = zeros
        return _

    lax.fori_loop(0, NODES_PAD // L, zero_body, None)

    ebase = s * EDGES_PER_TILE
    pltpu.sync_copy(edges_hbm.at[0, pl.ds(ebase, EDGES_PER_TILE)], ebuf0)
    pltpu.sync_copy(edges_hbm.at[1, pl.ds(ebase, EDGES_PER_TILE)], ebuf1)

    def edge_body(i, _):
        src = ebuf0[pl.ds(i * L, L)]
        plsc.addupdate_scatter(hist_out, [src], ones)
        dst = ebuf1[pl.ds(i * L, L)]
        plsc.addupdate_scatter(hist_in, [dst], ones)
        return _

    lax.fori_loop(0, EDGES_PER_TILE // L, edge_body, None)

    pltpu.sync_copy(hist_in, shared.at[0, s])
    pltpu.sync_copy(hist_out, shared.at[1, s])
    plsc.subcore_barrier()

    w = c * NS + s
    gbase = w * NODES_PER_TILE
    zmax = jnp.full((L,), Z_ROWS - 1, jnp.int32)

    def combine(which, idx_ref):
        pltpu.sync_copy(shared.at[which, :, pl.ds(gbase, NODES_PER_TILE)], cbuf)

        def comb_body(j, _):
            acc = cbuf[0, pl.ds(j * L, L)]
            for r in range(1, NS):
                acc = acc + cbuf[r, pl.ds(j * L, L)]
            idx_ref[pl.ds(j * L, L)] = jnp.minimum(acc, zmax)
            return _

        lax.fori_loop(0, NODES_PER_TILE // L, comb_body, None)

    combine(0, idx_in)
    combine(1, idx_out)

    for k in range(N_SUB):
        nbase = gbase + k * SUB

        @pl.when(nbase < N_NODES)
        def _():
            cp_x = pltpu.async_copy(x_hbm.at[pl.ds(nbase, SUB)], xbuf, sem_x)
            cp_zi = pltpu.async_copy(
                zin_hbm.at[idx_in.at[pl.ds(k * SUB, SUB)]], zin_buf, sem_zi)
            cp_zo = pltpu.async_copy(
                zout_hbm.at[idx_out.at[pl.ds(k * SUB, SUB)]], zout_buf, sem_zo)
            cp_x.wait()
            cp_zi.wait()
            cp_zo.wait()

            def add_body(r, _):
                for cc in range(NODE_DIM // L):
                    sl = pl.ds(cc * L, L)
                    xbuf[r, sl] = xbuf[r, sl] + zin_buf[r, sl] + zout_buf[r, sl]
                return _

            lax.fori_loop(0, SUB, add_body, None)
            pltpu.sync_copy(xbuf, out_hbm.at[pl.ds(nbase, SUB)])


@jax.jit
def _centrality(x, edge_index, z_in, z_out):
    mesh = plsc.VectorSubcoreMesh(core_axis_name="c", subcore_axis_name="s")
    run = functools.partial(
        pl.kernel,
        out_type=jax.ShapeDtypeStruct((N_NODES, NODE_DIM), jnp.float32),
        mesh=mesh,
        scratch_types=[
            pltpu.VMEM((EDGES_PER_TILE,), jnp.int32),
            pltpu.VMEM((EDGES_PER_TILE,), jnp.int32),
            pltpu.VMEM((NODES_PAD,), jnp.int32),
            pltpu.VMEM((NODES_PAD,), jnp.int32),
            pltpu.VMEM_SHARED((2, NS, NODES_PAD), jnp.int32),
            pltpu.VMEM((NS, NODES_PER_TILE), jnp.int32),
            pltpu.VMEM((NODES_PER_TILE,), jnp.int32),
            pltpu.VMEM((NODES_PER_TILE,), jnp.int32),
            pltpu.VMEM((SUB, NODE_DIM), jnp.float32),
            pltpu.VMEM((SUB, NODE_DIM), jnp.float32),
            pltpu.VMEM((SUB, NODE_DIM), jnp.float32),
            pltpu.SemaphoreType.DMA,
            pltpu.SemaphoreType.DMA,
            pltpu.SemaphoreType.DMA,
        ],
    )(_body)
    return run(x, edge_index, z_in, z_out)


def kernel(x, edge_index, z_in, z_out):
    return _centrality(x, edge_index.astype(jnp.int32), z_in, z_out)


# speedup vs baseline: 1.2777x; 1.2777x over previous
import functools

import jax
import jax.numpy as jnp
from jax import lax
from jax.experimental import pallas as pl
from jax.experimental.pallas import tpu as pltpu
from jax.experimental.pallas import tpu_sc as plsc

N_NODES = 10000
N_EDGES = 320000
NODE_DIM = 128
Z_ROWS = 256

NC = 2
NS = 16
L = 16

NODES_PAD = 10240
NODES_PER_TILE = NODES_PAD // (NC * NS)
SUB = 80
N_SUB = NODES_PER_TILE // SUB
EDGES_PER_TILE = N_EDGES // NS


def _body(x_hbm, edges_hbm, zin_hbm, zout_hbm, out_hbm,
          ebuf0, ebuf1, hist_in, hist_out, shared, cbuf,
          idx_in, idx_out, xbuf, zin_buf, zout_buf,
          sem_x, sem_zi, sem_zo):
    c = lax.axis_index("c")
    s = lax.axis_index("s")

    zeros = jnp.zeros((L,), jnp.int32)
    ones = jnp.ones((L,), jnp.int32)

    def zero_body(i, _):
        hist_in[pl.ds(i * L, L)] = zeros
        hist_out[pl.ds(i * L, L)] = zeros
        return _

    lax.fori_loop(0, NODES_PAD // L, zero_body, None)

    ebase = s * EDGES_PER_TILE
    pltpu.sync_copy(edges_hbm.at[pl.ds(ebase, EDGES_PER_TILE)], ebuf0)
    pltpu.sync_copy(edges_hbm.at[pl.ds(N_EDGES + ebase, EDGES_PER_TILE)], ebuf1)

    def edge_body(i, _):
        src = ebuf0[pl.ds(i * L, L)]
        plsc.addupdate_scatter(hist_out, [src], ones)
        dst = ebuf1[pl.ds(i * L, L)]
        plsc.addupdate_scatter(hist_in, [dst], ones)
        return _

    lax.fori_loop(0, EDGES_PER_TILE // L, edge_body, None)

    pltpu.sync_copy(hist_in, shared.at[pl.ds(s * NODES_PAD, NODES_PAD)])
    pltpu.sync_copy(
        hist_out, shared.at[pl.ds((NS + s) * NODES_PAD, NODES_PAD)])
    plsc.subcore_barrier()

    w = c * NS + s
    gbase = w * NODES_PER_TILE
    zmax = jnp.full((L,), Z_ROWS - 1, jnp.int32)

    def combine(which, idx_ref):
        for r in range(NS):
            pltpu.sync_copy(
                shared.at[pl.ds((which * NS + r) * NODES_PAD + gbase,
                                NODES_PER_TILE)],
                cbuf.at[pl.ds(r * NODES_PER_TILE, NODES_PER_TILE)])

        def comb_body(j, _):
            acc = cbuf[pl.ds(j * L, L)]
            for r in range(1, NS):
                acc = acc + cbuf[pl.ds(r * NODES_PER_TILE + j * L, L)]
            idx_ref[pl.ds(j * L, L)] = jnp.minimum(acc, zmax)
            return _

        lax.fori_loop(0, NODES_PER_TILE // L, comb_body, None)

    combine(0, idx_in)
    combine(1, idx_out)

    for k in range(N_SUB):
        nbase = gbase + k * SUB

        @pl.when(nbase < N_NODES)
        def _():
            cp_x = pltpu.async_copy(x_hbm.at[pl.ds(nbase, SUB)], xbuf, sem_x)
            cp_zi = pltpu.async_copy(
                zin_hbm.at[idx_in.at[pl.ds(k * SUB, SUB)]], zin_buf, sem_zi)
            cp_zo = pltpu.async_copy(
                zout_hbm.at[idx_out.at[pl.ds(k * SUB, SUB)]], zout_buf, sem_zo)
            cp_x.wait()
            cp_zi.wait()
            cp_zo.wait()

            def add_body(r, _):
                for cc in range(NODE_DIM // L):
                    sl = pl.ds(cc * L, L)
                    xbuf[r, sl] = xbuf[r, sl] + zin_buf[r, sl] + zout_buf[r, sl]
                return _

            lax.fori_loop(0, SUB, add_body, None)
            pltpu.sync_copy(xbuf, out_hbm.at[pl.ds(nbase, SUB)])


@jax.jit
def _centrality(x, edge_index, z_in, z_out):
    mesh = plsc.VectorSubcoreMesh(core_axis_name="c", subcore_axis_name="s")
    run = functools.partial(
        pl.kernel,
        out_type=jax.ShapeDtypeStruct((N_NODES, NODE_DIM), jnp.float32),
        mesh=mesh,
        compiler_params=pltpu.CompilerParams(needs_layout_passes=False),
        scratch_types=[
            pltpu.VMEM((EDGES_PER_TILE,), jnp.int32),
            pltpu.VMEM((EDGES_PER_TILE,), jnp.int32),
            pltpu.VMEM((NODES_PAD,), jnp.int32),
            pltpu.VMEM((NODES_PAD,), jnp.int32),
            pltpu.VMEM_SHARED((2 * NS * NODES_PAD,), jnp.int32),
            pltpu.VMEM((NS * NODES_PER_TILE,), jnp.int32),
            pltpu.VMEM((NODES_PER_TILE,), jnp.int32),
            pltpu.VMEM((NODES_PER_TILE,), jnp.int32),
            pltpu.VMEM((SUB, NODE_DIM), jnp.float32),
            pltpu.VMEM((SUB, NODE_DIM), jnp.float32),
            pltpu.VMEM((SUB, NODE_DIM), jnp.float32),
            pltpu.SemaphoreType.DMA,
            pltpu.SemaphoreType.DMA,
            pltpu.SemaphoreType.DMA,
        ],
    )(_body)
    return run(x, edge_index, z_in, z_out)


def kernel(x, edge_index, z_in, z_out):
    edges_flat = edge_index.astype(jnp.int32).reshape(-1)
    return _centrality(x, edges_flat, z_in, z_out)


# speedup vs baseline: 1.3749x; 1.0761x over previous
import functools

import jax
import jax.numpy as jnp
from jax import lax
from jax.experimental import pallas as pl
from jax.experimental.pallas import tpu as pltpu
from jax.experimental.pallas import tpu_sc as plsc

N_NODES = 10000
N_EDGES = 320000
NODE_DIM = 128
Z_ROWS = 256

NC = 2
NS = 16
L = 16

NODES_PAD = 10240
NODES_PER_TILE = NODES_PAD // (NC * NS)
SUB = 40
N_SUB = NODES_PER_TILE // SUB
EDGES_PER_TILE = N_EDGES // NS
UNROLL = 5


def _body(x_hbm, edges_hbm, zin_hbm, zout_hbm, out_hbm,
          ebuf0, ebuf1, hist_in, hist_out, shared, cbuf,
          idx_in, idx_out, xb0, xb1, zib0, zib1, zob0, zob1,
          sem_e0, sem_e1, sem_c,
          sem_x0, sem_x1, sem_zi0, sem_zi1, sem_zo0, sem_zo1,
          sem_st0, sem_st1):
    c = lax.axis_index("c")
    s = lax.axis_index("s")

    zeros = jnp.zeros((L,), jnp.int32)
    ones = jnp.ones((L,), jnp.int32)

    ebase = s * EDGES_PER_TILE
    cp_e0 = pltpu.async_copy(
        edges_hbm.at[pl.ds(ebase, EDGES_PER_TILE)], ebuf0, sem_e0)
    cp_e1 = pltpu.async_copy(
        edges_hbm.at[pl.ds(N_EDGES + ebase, EDGES_PER_TILE)], ebuf1, sem_e1)

    def zero_body(i, _):
        for u in range(8):
            hist_in[pl.ds((i * 8 + u) * L, L)] = zeros
            hist_out[pl.ds((i * 8 + u) * L, L)] = zeros
        return _

    lax.fori_loop(0, NODES_PAD // (8 * L), zero_body, None)
    cp_e0.wait()
    cp_e1.wait()

    def edge_body(i, _):
        for u in range(UNROLL):
            off = (i * UNROLL + u) * L
            src = ebuf0[pl.ds(off, L)]
            plsc.addupdate_scatter(hist_out, [src], ones)
            dst = ebuf1[pl.ds(off, L)]
            plsc.addupdate_scatter(hist_in, [dst], ones)
        return _

    lax.fori_loop(0, EDGES_PER_TILE // (L * UNROLL), edge_body, None)

    pltpu.sync_copy(hist_in, shared.at[pl.ds(s * NODES_PAD, NODES_PAD)])
    pltpu.sync_copy(
        hist_out, shared.at[pl.ds((NS + s) * NODES_PAD, NODES_PAD)])
    plsc.subcore_barrier()

    w = c * NS + s
    gbase = w * NODES_PER_TILE
    zmax = jnp.full((L,), Z_ROWS - 1, jnp.int32)

    cps = []
    for which in range(2):
        for r in range(NS):
            cps.append(pltpu.async_copy(
                shared.at[pl.ds((which * NS + r) * NODES_PAD + gbase,
                                NODES_PER_TILE)],
                cbuf.at[pl.ds((which * NS + r) * NODES_PER_TILE,
                              NODES_PER_TILE)],
                sem_c))
    for cp in cps:
        cp.wait()

    def combine(which, idx_ref):
        def comb_body(j, _):
            base = which * NS * NODES_PER_TILE
            acc = cbuf[pl.ds(base + j * L, L)]
            for r in range(1, NS):
                acc = acc + cbuf[pl.ds(base + r * NODES_PER_TILE + j * L, L)]
            idx_ref[pl.ds(j * L, L)] = jnp.minimum(acc, zmax)
            return _

        lax.fori_loop(0, NODES_PER_TILE // L, comb_body, None)

    combine(0, idx_in)
    combine(1, idx_out)

    xb = (xb0, xb1)
    zib = (zib0, zib1)
    zob = (zob0, zob1)
    sem_x = (sem_x0, sem_x1)
    sem_zi = (sem_zi0, sem_zi1)
    sem_zo = (sem_zo0, sem_zo1)
    sem_st = (sem_st0, sem_st1)

    def issue(k):
        b = k % 2
        nbase = gbase + k * SUB

        @pl.when(nbase < N_NODES)
        def _():
            if k >= 2:
                pltpu.make_async_copy(
                    xb[b], out_hbm.at[pl.ds(gbase + (k - 2) * SUB, SUB)],
                    sem_st[b]).wait()
            pltpu.async_copy(x_hbm.at[pl.ds(nbase, SUB)], xb[b], sem_x[b])
            pltpu.async_copy(
                zin_hbm.at[idx_in.at[pl.ds(k * SUB, SUB)]], zib[b], sem_zi[b])
            pltpu.async_copy(
                zout_hbm.at[idx_out.at[pl.ds(k * SUB, SUB)]], zob[b],
                sem_zo[b])

    def process(k):
        b = k % 2
        nbase = gbase + k * SUB

        @pl.when(nbase < N_NODES)
        def _():
            pltpu.make_async_copy(
                x_hbm.at[pl.ds(nbase, SUB)], xb[b], sem_x[b]).wait()
            pltpu.make_async_copy(
                zin_hbm.at[idx_in.at[pl.ds(k * SUB, SUB)]], zib[b],
                sem_zi[b]).wait()
            pltpu.make_async_copy(
                zout_hbm.at[idx_out.at[pl.ds(k * SUB, SUB)]], zob[b],
                sem_zo[b]).wait()

            def add_body(r, _):
                for cc in range(NODE_DIM // L):
                    sl = pl.ds(cc * L, L)
                    xb[b][r, sl] = xb[b][r, sl] + zib[b][r, sl] + zob[b][r, sl]
                return _

            lax.fori_loop(0, SUB, add_body, None)
            pltpu.async_copy(xb[b], out_hbm.at[pl.ds(nbase, SUB)], sem_st[b])

    issue(0)
    for k in range(N_SUB):
        if k + 1 < N_SUB:
            issue(k + 1)
        process(k)

    for k in (N_SUB - 2, N_SUB - 1):
        b = k % 2
        nbase = gbase + k * SUB

        @pl.when(nbase < N_NODES)
        def _():
            pltpu.make_async_copy(
                xb[b], out_hbm.at[pl.ds(nbase, SUB)], sem_st[b]).wait()


@jax.jit
def _centrality(x, edge_index, z_in, z_out):
    mesh = plsc.VectorSubcoreMesh(core_axis_name="c", subcore_axis_name="s")
    run = functools.partial(
        pl.kernel,
        out_type=jax.ShapeDtypeStruct((N_NODES, NODE_DIM), jnp.float32),
        mesh=mesh,
        compiler_params=pltpu.CompilerParams(needs_layout_passes=False),
        scratch_types=[
            pltpu.VMEM((EDGES_PER_TILE,), jnp.int32),
            pltpu.VMEM((EDGES_PER_TILE,), jnp.int32),
            pltpu.VMEM((NODES_PAD,), jnp.int32),
            pltpu.VMEM((NODES_PAD,), jnp.int32),
            pltpu.VMEM_SHARED((2 * NS * NODES_PAD,), jnp.int32),
            pltpu.VMEM((2 * NS * NODES_PER_TILE,), jnp.int32),
            pltpu.VMEM((NODES_PER_TILE,), jnp.int32),
            pltpu.VMEM((NODES_PER_TILE,), jnp.int32),
            pltpu.VMEM((SUB, NODE_DIM), jnp.float32),
            pltpu.VMEM((SUB, NODE_DIM), jnp.float32),
            pltpu.VMEM((SUB, NODE_DIM), jnp.float32),
            pltpu.VMEM((SUB, NODE_DIM), jnp.float32),
            pltpu.VMEM((SUB, NODE_DIM), jnp.float32),
            pltpu.VMEM((SUB, NODE_DIM), jnp.float32),
        ] + [pltpu.SemaphoreType.DMA] * 11,
    )(_body)
    return run(x, edge_index, z_in, z_out)


def kernel(x, edge_index, z_in, z_out):
    edges_flat = edge_index.astype(jnp.int32).reshape(-1)
    return _centrality(x, edges_flat, z_in, z_out)


# speedup vs baseline: 1.5747x; 1.1453x over previous
import functools

import jax
import jax.numpy as jnp
from jax import lax
from jax.experimental import pallas as pl
from jax.experimental.pallas import tpu as pltpu
from jax.experimental.pallas import tpu_sc as plsc

N_NODES = 10000
N_EDGES = 320000
NODE_DIM = 128
Z_ROWS = 256

NC = 2
NS = 16
L = 16

NODES_PAD = 10240
NODES_PER_TILE = NODES_PAD // (NC * NS)
SUB = 40
N_SUB = NODES_PER_TILE // SUB
EDGES_PER_TILE = N_EDGES // NS
UNROLL = 8


def _body(x_hbm, edges_hbm, zin_hbm, zout_hbm, out_hbm,
          ebuf0, ebuf1, hist_in, hist_out, shared, cbuf,
          idx_in, idx_out, xb0, xb1, zib0, zib1, zob0, zob1,
          sem_e0, sem_e1, sem_c,
          sem_x0, sem_x1, sem_zi0, sem_zi1, sem_zo0, sem_zo1,
          sem_st0, sem_st1):
    c = lax.axis_index("c")
    s = lax.axis_index("s")

    zeros = jnp.zeros((L,), jnp.int32)
    ones = jnp.ones((L,), jnp.int32)

    ebase = s * EDGES_PER_TILE
    cp_e0 = pltpu.async_copy(
        edges_hbm.at[pl.ds(ebase, EDGES_PER_TILE)], ebuf0, sem_e0)
    cp_e1 = pltpu.async_copy(
        edges_hbm.at[pl.ds(N_EDGES + ebase, EDGES_PER_TILE)], ebuf1, sem_e1)

    def zero_body(i, _):
        for u in range(8):
            hist_in[pl.ds((i * 8 + u) * L, L)] = zeros
            hist_out[pl.ds((i * 8 + u) * L, L)] = zeros
        return _

    lax.fori_loop(0, NODES_PAD // (8 * L), zero_body, None)
    cp_e0.wait()
    cp_e1.wait()

    def edge_body(i, _):
        offs = [(i * UNROLL + u) * L for u in range(UNROLL)]
        srcs = [ebuf0[pl.ds(o, L)] for o in offs]
        dsts = [ebuf1[pl.ds(o, L)] for o in offs]
        for u in range(UNROLL):
            plsc.addupdate_scatter(hist_out, [srcs[u]], ones)
            plsc.addupdate_scatter(hist_in, [dsts[u]], ones)
        return _

    lax.fori_loop(0, EDGES_PER_TILE // (L * UNROLL), edge_body, None)

    pltpu.sync_copy(hist_in, shared.at[pl.ds(s * NODES_PAD, NODES_PAD)])
    pltpu.sync_copy(
        hist_out, shared.at[pl.ds((NS + s) * NODES_PAD, NODES_PAD)])
    plsc.subcore_barrier()

    w = c * NS + s
    gbase = w * NODES_PER_TILE
    zmax = jnp.full((L,), Z_ROWS - 1, jnp.int32)

    cps = []
    for which in range(2):
        for r in range(NS):
            cps.append(pltpu.async_copy(
                shared.at[pl.ds((which * NS + r) * NODES_PAD + gbase,
                                NODES_PER_TILE)],
                cbuf.at[pl.ds((which * NS + r) * NODES_PER_TILE,
                              NODES_PER_TILE)],
                sem_c))
    for cp in cps:
        cp.wait()

    def combine(which, idx_ref):
        def comb_body(j, _):
            base = which * NS * NODES_PER_TILE
            acc = cbuf[pl.ds(base + j * L, L)]
            for r in range(1, NS):
                acc = acc + cbuf[pl.ds(base + r * NODES_PER_TILE + j * L, L)]
            idx_ref[pl.ds(j * L, L)] = jnp.minimum(acc, zmax)
            return _

        lax.fori_loop(0, NODES_PER_TILE // L, comb_body, None)

    combine(0, idx_in)
    combine(1, idx_out)

    xb = (xb0, xb1)
    zib = (zib0, zib1)
    zob = (zob0, zob1)
    sem_x = (sem_x0, sem_x1)
    sem_zi = (sem_zi0, sem_zi1)
    sem_zo = (sem_zo0, sem_zo1)
    sem_st = (sem_st0, sem_st1)

    def issue(k):
        b = k % 2
        nbase = gbase + k * SUB

        @pl.when(nbase < N_NODES)
        def _():
            if k >= 2:
                pltpu.make_async_copy(
                    xb[b], out_hbm.at[pl.ds(gbase + (k - 2) * SUB, SUB)],
                    sem_st[b]).wait()
            pltpu.async_copy(x_hbm.at[pl.ds(nbase, SUB)], xb[b], sem_x[b])
            pltpu.async_copy(
                zin_hbm.at[idx_in.at[pl.ds(k * SUB, SUB)]], zib[b], sem_zi[b])
            pltpu.async_copy(
                zout_hbm.at[idx_out.at[pl.ds(k * SUB, SUB)]], zob[b],
                sem_zo[b])

    def process(k):
        b = k % 2
        nbase = gbase + k * SUB

        @pl.when(nbase < N_NODES)
        def _():
            pltpu.make_async_copy(
                x_hbm.at[pl.ds(nbase, SUB)], xb[b], sem_x[b]).wait()
            pltpu.make_async_copy(
                zin_hbm.at[idx_in.at[pl.ds(k * SUB, SUB)]], zib[b],
                sem_zi[b]).wait()
            pltpu.make_async_copy(
                zout_hbm.at[idx_out.at[pl.ds(k * SUB, SUB)]], zob[b],
                sem_zo[b]).wait()

            def add_body(r, _):
                for cc in range(NODE_DIM // L):
                    sl = pl.ds(cc * L, L)
                    xb[b][r, sl] = xb[b][r, sl] + zib[b][r, sl] + zob[b][r, sl]
                return _

            lax.fori_loop(0, SUB, add_body, None)
            pltpu.async_copy(xb[b], out_hbm.at[pl.ds(nbase, SUB)], sem_st[b])

    issue(0)
    for k in range(N_SUB):
        if k + 1 < N_SUB:
            issue(k + 1)
        process(k)

    for k in (N_SUB - 2, N_SUB - 1):
        b = k % 2
        nbase = gbase + k * SUB

        @pl.when(nbase < N_NODES)
        def _():
            pltpu.make_async_copy(
                xb[b], out_hbm.at[pl.ds(nbase, SUB)], sem_st[b]).wait()


@jax.jit
def _centrality(x, edge_index, z_in, z_out):
    mesh = plsc.VectorSubcoreMesh(core_axis_name="c", subcore_axis_name="s")
    run = functools.partial(
        pl.kernel,
        out_type=jax.ShapeDtypeStruct((N_NODES, NODE_DIM), jnp.float32),
        mesh=mesh,
        compiler_params=pltpu.CompilerParams(needs_layout_passes=False),
        scratch_types=[
            pltpu.VMEM((EDGES_PER_TILE,), jnp.int32),
            pltpu.VMEM((EDGES_PER_TILE,), jnp.int32),
            pltpu.VMEM((NODES_PAD,), jnp.int32),
            pltpu.VMEM((NODES_PAD,), jnp.int32),
            pltpu.VMEM_SHARED((2 * NS * NODES_PAD,), jnp.int32),
            pltpu.VMEM((2 * NS * NODES_PER_TILE,), jnp.int32),
            pltpu.VMEM((NODES_PER_TILE,), jnp.int32),
            pltpu.VMEM((NODES_PER_TILE,), jnp.int32),
            pltpu.VMEM((SUB, NODE_DIM), jnp.float32),
            pltpu.VMEM((SUB, NODE_DIM), jnp.float32),
            pltpu.VMEM((SUB, NODE_DIM), jnp.float32),
            pltpu.VMEM((SUB, NODE_DIM), jnp.float32),
            pltpu.VMEM((SUB, NODE_DIM), jnp.float32),
            pltpu.VMEM((SUB, NODE_DIM), jnp.float32),
        ] + [pltpu.SemaphoreType.DMA] * 11,
    )(_body)
    return run(x, edge_index, z_in, z_out)


def kernel(x, edge_index, z_in, z_out):
    edges_flat = edge_index.astype(jnp.int32).reshape(-1)
    return _centrality(x, edges_flat, z_in, z_out)


# speedup vs baseline: 1.5918x; 1.0109x over previous
import functools

import jax
import jax.numpy as jnp
from jax import lax
from jax.experimental import pallas as pl
from jax.experimental.pallas import tpu as pltpu
from jax.experimental.pallas import tpu_sc as plsc

N_NODES = 10000
N_EDGES = 320000
NODE_DIM = 128
Z_ROWS = 256

NC = 2
NS = 16
L = 16

NODES_PAD = 10240
NODES_PER_TILE = NODES_PAD // (NC * NS)
SUB = 40
N_SUB = NODES_PER_TILE // SUB
EDGES_PER_TILE = N_EDGES // NS
UNROLL = 10


def _body(x_hbm, edges_hbm, zin_hbm, zout_hbm, out_hbm,
          ebuf0, ebuf1, hist_in, hist_out, shared, cbuf,
          idx_in, idx_out, xb0, xb1, zib0, zib1, zob0, zob1,
          sem_e0, sem_e1, sem_c,
          sem_x0, sem_x1, sem_zi0, sem_zi1, sem_zo0, sem_zo1,
          sem_st0, sem_st1):
    c = lax.axis_index("c")
    s = lax.axis_index("s")

    zeros = jnp.zeros((L,), jnp.int32)
    ones = jnp.ones((L,), jnp.int32)

    ebase = s * EDGES_PER_TILE
    cp_e0 = pltpu.async_copy(
        edges_hbm.at[pl.ds(ebase, EDGES_PER_TILE)], ebuf0, sem_e0)
    cp_e1 = pltpu.async_copy(
        edges_hbm.at[pl.ds(N_EDGES + ebase, EDGES_PER_TILE)], ebuf1, sem_e1)

    def zero_body(i, _):
        for u in range(8):
            hist_in[pl.ds((i * 8 + u) * L, L)] = zeros
            hist_out[pl.ds((i * 8 + u) * L, L)] = zeros
        return _

    lax.fori_loop(0, NODES_PAD // (8 * L), zero_body, None)
    cp_e0.wait()
    cp_e1.wait()

    def edge_body(i, _):
        offs = [(i * UNROLL + u) * L for u in range(UNROLL)]
        srcs = [ebuf0[pl.ds(o, L)] for o in offs]
        dsts = [ebuf1[pl.ds(o, L)] for o in offs]
        for u in range(UNROLL):
            plsc.addupdate_scatter(hist_out, [srcs[u]], ones)
            plsc.addupdate_scatter(hist_in, [dsts[u]], ones)
        return _

    lax.fori_loop(0, EDGES_PER_TILE // (L * UNROLL), edge_body, None)

    pltpu.sync_copy(hist_in, shared.at[pl.ds(s * NODES_PAD, NODES_PAD)])
    pltpu.sync_copy(
        hist_out, shared.at[pl.ds((NS + s) * NODES_PAD, NODES_PAD)])
    plsc.subcore_barrier()

    w = c * NS + s
    gbase = w * NODES_PER_TILE
    zmax = jnp.full((L,), Z_ROWS - 1, jnp.int32)

    cps = []
    for which in range(2):
        for r in range(NS):
            cps.append(pltpu.async_copy(
                shared.at[pl.ds((which * NS + r) * NODES_PAD + gbase,
                                NODES_PER_TILE)],
                cbuf.at[pl.ds((which * NS + r) * NODES_PER_TILE,
                              NODES_PER_TILE)],
                sem_c))
    for cp in cps:
        cp.wait()

    def combine(which, idx_ref):
        def comb_body(j, _):
            base = which * NS * NODES_PER_TILE
            acc = cbuf[pl.ds(base + j * L, L)]
            for r in range(1, NS):
                acc = acc + cbuf[pl.ds(base + r * NODES_PER_TILE + j * L, L)]
            idx_ref[pl.ds(j * L, L)] = jnp.minimum(acc, zmax)
            return _

        lax.fori_loop(0, NODES_PER_TILE // L, comb_body, None)

    combine(0, idx_in)
    combine(1, idx_out)

    xb = (xb0, xb1)
    zib = (zib0, zib1)
    zob = (zob0, zob1)
    sem_x = (sem_x0, sem_x1)
    sem_zi = (sem_zi0, sem_zi1)
    sem_zo = (sem_zo0, sem_zo1)
    sem_st = (sem_st0, sem_st1)

    def issue(k):
        b = k % 2
        nbase = gbase + k * SUB

        @pl.when(nbase < N_NODES)
        def _():
            if k >= 2:
                pltpu.make_async_copy(
                    xb[b], out_hbm.at[pl.ds(gbase + (k - 2) * SUB, SUB)],
                    sem_st[b]).wait()
            pltpu.async_copy(x_hbm.at[pl.ds(nbase, SUB)], xb[b], sem_x[b])
            pltpu.async_copy(
                zin_hbm.at[idx_in.at[pl.ds(k * SUB, SUB)]], zib[b], sem_zi[b])
            pltpu.async_copy(
                zout_hbm.at[idx_out.at[pl.ds(k * SUB, SUB)]], zob[b],
                sem_zo[b])

    def process(k):
        b = k % 2
        nbase = gbase + k * SUB

        @pl.when(nbase < N_NODES)
        def _():
            pltpu.make_async_copy(
                x_hbm.at[pl.ds(nbase, SUB)], xb[b], sem_x[b]).wait()
            pltpu.make_async_copy(
                zin_hbm.at[idx_in.at[pl.ds(k * SUB, SUB)]], zib[b],
                sem_zi[b]).wait()
            pltpu.make_async_copy(
                zout_hbm.at[idx_out.at[pl.ds(k * SUB, SUB)]], zob[b],
                sem_zo[b]).wait()

            def add_body(r, _):
                for cc in range(NODE_DIM // L):
                    sl = pl.ds(cc * L, L)
                    xb[b][r, sl] = xb[b][r, sl] + zib[b][r, sl] + zob[b][r, sl]
                return _

            lax.fori_loop(0, SUB, add_body, None)
            pltpu.async_copy(xb[b], out_hbm.at[pl.ds(nbase, SUB)], sem_st[b])

    issue(0)
    for k in range(N_SUB):
        if k + 1 < N_SUB:
            issue(k + 1)
        process(k)

    for k in (N_SUB - 2, N_SUB - 1):
        b = k % 2
        nbase = gbase + k * SUB

        @pl.when(nbase < N_NODES)
        def _():
            pltpu.make_async_copy(
                xb[b], out_hbm.at[pl.ds(nbase, SUB)], sem_st[b]).wait()


@jax.jit
def _centrality(x, edge_index, z_in, z_out):
    mesh = plsc.VectorSubcoreMesh(core_axis_name="c", subcore_axis_name="s")
    run = functools.partial(
        pl.kernel,
        out_type=jax.ShapeDtypeStruct((N_NODES, NODE_DIM), jnp.float32),
        mesh=mesh,
        compiler_params=pltpu.CompilerParams(needs_layout_passes=False),
        scratch_types=[
            pltpu.VMEM((EDGES_PER_TILE,), jnp.int32),
            pltpu.VMEM((EDGES_PER_TILE,), jnp.int32),
            pltpu.VMEM((NODES_PAD,), jnp.int32),
            pltpu.VMEM((NODES_PAD,), jnp.int32),
            pltpu.VMEM_SHARED((2 * NS * NODES_PAD,), jnp.int32),
            pltpu.VMEM((2 * NS * NODES_PER_TILE,), jnp.int32),
            pltpu.VMEM((NODES_PER_TILE,), jnp.int32),
            pltpu.VMEM((NODES_PER_TILE,), jnp.int32),
            pltpu.VMEM((SUB, NODE_DIM), jnp.float32),
            pltpu.VMEM((SUB, NODE_DIM), jnp.float32),
            pltpu.VMEM((SUB, NODE_DIM), jnp.float32),
            pltpu.VMEM((SUB, NODE_DIM), jnp.float32),
            pltpu.VMEM((SUB, NODE_DIM), jnp.float32),
            pltpu.VMEM((SUB, NODE_DIM), jnp.float32),
        ] + [pltpu.SemaphoreType.DMA] * 11,
    )(_body)
    return run(x, edge_index, z_in, z_out)


def kernel(x, edge_index, z_in, z_out):
    edges_flat = edge_index.astype(jnp.int32).reshape(-1)
    return _centrality(x, edges_flat, z_in, z_out)


# speedup vs baseline: 1.5978x; 1.0038x over previous
import functools

import jax
import jax.numpy as jnp
from jax import lax
from jax.experimental import pallas as pl
from jax.experimental.pallas import tpu as pltpu
from jax.experimental.pallas import tpu_sc as plsc

N_NODES = 10000
N_EDGES = 320000
NODE_DIM = 128
Z_ROWS = 256

NC = 2
NS = 16
L = 16

NODES_PAD = 10240
NODES_PER_TILE = NODES_PAD // (NC * NS)
SUB = 40
N_SUB = NODES_PER_TILE // SUB
EDGES_PER_TILE = N_EDGES // NS
UNROLL = 10


def _body(x_hbm, edges_hbm, zin_hbm, zout_hbm, out_hbm,
          ebuf0, ebuf1, hist_in, hist_out, shared, cbuf,
          idx_in, idx_out, xb0, xb1, zib0, zib1, zob0, zob1,
          sem_e0, sem_e1, sem_c,
          sem_x0, sem_x1, sem_zi0, sem_zi1, sem_zo0, sem_zo1,
          sem_st0, sem_st1):
    c = lax.axis_index("c")
    s = lax.axis_index("s")

    zeros = jnp.zeros((L,), jnp.int32)
    ones = jnp.ones((L,), jnp.int32)

    ebase = s * EDGES_PER_TILE
    cp_e0 = pltpu.async_copy(
        edges_hbm.at[pl.ds(ebase, EDGES_PER_TILE)], ebuf0, sem_e0)
    cp_e1 = pltpu.async_copy(
        edges_hbm.at[pl.ds(N_EDGES + ebase, EDGES_PER_TILE)], ebuf1, sem_e1)

    with jax.named_scope("p1_zero"):
        def zero_body(i, _):
            for u in range(8):
                hist_in[pl.ds((i * 8 + u) * L, L)] = zeros
                hist_out[pl.ds((i * 8 + u) * L, L)] = zeros
            return _

        lax.fori_loop(0, NODES_PAD // (8 * L), zero_body, None)
    with jax.named_scope("p1_ewait"):
        cp_e0.wait()
        cp_e1.wait()

    def edge_body(i, _):
        offs = [(i * UNROLL + u) * L for u in range(UNROLL)]
        srcs = [ebuf0[pl.ds(o, L)] for o in offs]
        dsts = [ebuf1[pl.ds(o, L)] for o in offs]
        for u in range(UNROLL):
            plsc.addupdate_scatter(hist_out, [srcs[u]], ones)
            plsc.addupdate_scatter(hist_in, [dsts[u]], ones)
        return _

    with jax.named_scope("p1_scatter"):
        lax.fori_loop(0, EDGES_PER_TILE // (L * UNROLL), edge_body, None)

    with jax.named_scope("c_publish"):
        pltpu.sync_copy(hist_in, shared.at[pl.ds(s * NODES_PAD, NODES_PAD)])
        pltpu.sync_copy(
            hist_out, shared.at[pl.ds((NS + s) * NODES_PAD, NODES_PAD)])
    with jax.named_scope("c_barrier"):
        plsc.subcore_barrier()

    w = c * NS + s
    gbase = w * NODES_PER_TILE
    zmax = jnp.full((L,), Z_ROWS - 1, jnp.int32)

    with jax.named_scope("c_read"):
        cps = []
        for which in range(2):
            for r in range(NS):
                cps.append(pltpu.async_copy(
                    shared.at[pl.ds((which * NS + r) * NODES_PAD + gbase,
                                    NODES_PER_TILE)],
                    cbuf.at[pl.ds((which * NS + r) * NODES_PER_TILE,
                                  NODES_PER_TILE)],
                    sem_c))
        for cp in cps:
            cp.wait()

    def combine(which, idx_ref):
        def comb_body(j, _):
            base = which * NS * NODES_PER_TILE
            acc = cbuf[pl.ds(base + j * L, L)]
            for r in range(1, NS):
                acc = acc + cbuf[pl.ds(base + r * NODES_PER_TILE + j * L, L)]
            idx_ref[pl.ds(j * L, L)] = jnp.minimum(acc, zmax)
            return _

        lax.fori_loop(0, NODES_PER_TILE // L, comb_body, None)

    with jax.named_scope("c_sum"):
        combine(0, idx_in)
        combine(1, idx_out)

    xb = (xb0, xb1)
    zib = (zib0, zib1)
    zob = (zob0, zob1)
    sem_x = (sem_x0, sem_x1)
    sem_zi = (sem_zi0, sem_zi1)
    sem_zo = (sem_zo0, sem_zo1)
    sem_st = (sem_st0, sem_st1)

    def issue(k):
        b = k % 2
        nbase = gbase + k * SUB

        @pl.when(nbase < N_NODES)
        def _():
            if k >= 2:
                pltpu.make_async_copy(
                    xb[b], out_hbm.at[pl.ds(gbase + (k - 2) * SUB, SUB)],
                    sem_st[b]).wait()
            pltpu.async_copy(x_hbm.at[pl.ds(nbase, SUB)], xb[b], sem_x[b])
            pltpu.async_copy(
                zin_hbm.at[idx_in.at[pl.ds(k * SUB, SUB)]], zib[b], sem_zi[b])
            pltpu.async_copy(
                zout_hbm.at[idx_out.at[pl.ds(k * SUB, SUB)]], zob[b],
                sem_zo[b])

    def process(k):
        b = k % 2
        nbase = gbase + k * SUB

        @pl.when(nbase < N_NODES)
        def _():
            pltpu.make_async_copy(
                x_hbm.at[pl.ds(nbase, SUB)], xb[b], sem_x[b]).wait()
            pltpu.make_async_copy(
                zin_hbm.at[idx_in.at[pl.ds(k * SUB, SUB)]], zib[b],
                sem_zi[b]).wait()
            pltpu.make_async_copy(
                zout_hbm.at[idx_out.at[pl.ds(k * SUB, SUB)]], zob[b],
                sem_zo[b]).wait()

            def add_body(r, _):
                for cc in range(NODE_DIM // L):
                    sl = pl.ds(cc * L, L)
                    xb[b][r, sl] = xb[b][r, sl] + zib[b][r, sl] + zob[b][r, sl]
                return _

            lax.fori_loop(0, SUB, add_body, None)
            pltpu.async_copy(xb[b], out_hbm.at[pl.ds(nbase, SUB)], sem_st[b])

    with jax.named_scope("p2"):
        issue(0)
        for k in range(N_SUB):
            if k + 1 < N_SUB:
                issue(k + 1)
            process(k)

    for k in (N_SUB - 2, N_SUB - 1):
        b = k % 2
        nbase = gbase + k * SUB

        @pl.when(nbase < N_NODES)
        def _():
            pltpu.make_async_copy(
                xb[b], out_hbm.at[pl.ds(nbase, SUB)], sem_st[b]).wait()


@jax.jit
def _centrality(x, edge_index, z_in, z_out):
    mesh = plsc.VectorSubcoreMesh(core_axis_name="c", subcore_axis_name="s")
    run = functools.partial(
        pl.kernel,
        out_type=jax.ShapeDtypeStruct((N_NODES, NODE_DIM), jnp.float32),
        mesh=mesh,
        compiler_params=pltpu.CompilerParams(needs_layout_passes=False),
        scratch_types=[
            pltpu.VMEM((EDGES_PER_TILE,), jnp.int32),
            pltpu.VMEM((EDGES_PER_TILE,), jnp.int32),
            pltpu.VMEM((NODES_PAD,), jnp.int32),
            pltpu.VMEM((NODES_PAD,), jnp.int32),
            pltpu.VMEM_SHARED((2 * NS * NODES_PAD,), jnp.int32),
            pltpu.VMEM((2 * NS * NODES_PER_TILE,), jnp.int32),
            pltpu.VMEM((NODES_PER_TILE,), jnp.int32),
            pltpu.VMEM((NODES_PER_TILE,), jnp.int32),
            pltpu.VMEM((SUB, NODE_DIM), jnp.float32),
            pltpu.VMEM((SUB, NODE_DIM), jnp.float32),
            pltpu.VMEM((SUB, NODE_DIM), jnp.float32),
            pltpu.VMEM((SUB, NODE_DIM), jnp.float32),
            pltpu.VMEM((SUB, NODE_DIM), jnp.float32),
            pltpu.VMEM((SUB, NODE_DIM), jnp.float32),
        ] + [pltpu.SemaphoreType.DMA] * 11,
    )(_body)
    return run(x, edge_index, z_in, z_out)


def kernel(x, edge_index, z_in, z_out):
    edges_flat = edge_index.astype(jnp.int32).reshape(-1)
    return _centrality(x, edges_flat, z_in, z_out)
